# baseline (speedup 1.0000x reference)
import functools

import jax
import jax.numpy as jnp
from jax import lax
from jax.experimental import pallas as pl
from jax.experimental.pallas import tpu as pltpu
from jax.experimental.pallas import tpu_sc as plsc

N = 10000
N_PAD = 10240
ROWS_PER_TILE = 640
ACC_ROWS = 10112
ACC_PER_TILE = 632
DUMMY = N
E = 320000
CHUNK = 128
NC, NS = 2, 16
NW = NC * NS
K0 = 103
K1 = 54
KT = K0 + K1
E_PAD = NS * KT * CHUNK
ZBASES = (0, 128, 256, 384, 504)
D_H = 128
D_OUT = 64

_mesh = plsc.VectorSubcoreMesh(core_axis_name="c", subcore_axis_name="s")


def _zero_vmem_2d(buf, rows, cols):
    def body(r, _):
        for k in range(cols // 16):
            buf[r, pl.ds(16 * k, 16)] = jnp.zeros((16,), jnp.float32)
        return 0
    lax.fori_loop(0, rows, body, 0)


@functools.partial(
    pl.kernel,
    out_type=jax.ShapeDtypeStruct((NC, 2, N_PAD), jnp.float32),
    mesh=_mesh,
    scratch_types=[
        pltpu.VMEM((KT, CHUNK), jnp.int32),
        pltpu.VMEM((KT, CHUNK), jnp.int32),
        pltpu.VMEM((CHUNK,), jnp.float32),
        pltpu.VMEM((ROWS_PER_TILE,), jnp.float32),
        pltpu.VMEM_SHARED((N_PAD,), jnp.float32),
        pltpu.VMEM_SHARED((N_PAD,), jnp.float32),
        pltpu.SemaphoreType.DMA((4,)),
        pltpu.SemaphoreType.DMA((4,)),
    ],
)
def _sc_degrees(src_hbm, dst_hbm, out_hbm, src_v, dst_v, ones_v, zero_v,
                acc_s, acc_d, ssot, ssod):
    c = lax.axis_index("c")
    s = lax.axis_index("s")
    for k in range(CHUNK // 16):
        ones_v[pl.ds(16 * k, 16)] = jnp.ones((16,), jnp.float32)
    if CHUNK % 16:
        ones_v[pl.ds(CHUNK - 16, 16)] = jnp.ones((16,), jnp.float32)
    for k in range(ROWS_PER_TILE // 16):
        zero_v[pl.ds(16 * k, 16)] = jnp.zeros((16,), jnp.float32)
    pltpu.sync_copy(zero_v, acc_s.at[pl.ds(s * ROWS_PER_TILE, ROWS_PER_TILE)])
    pltpu.sync_copy(zero_v, acc_d.at[pl.ds(s * ROWS_PER_TILE, ROWS_PER_TILE)])
    plsc.subcore_barrier()

    pltpu.sync_copy(src_hbm.at[s], src_v)
    pltpu.sync_copy(dst_hbm.at[s], dst_v)
    off = jnp.where(c == 0, 0, K0)
    kc = jnp.where(c == 0, K0, K1)

    def add(j, _):
        t = lax.rem(j, 4)
        r = j + off

        @pl.when(j >= 4)
        def _():
            pltpu.make_async_copy(
                ones_v, acc_s.at[src_v.at[r - 4]], ssot.at[t]).wait()
            pltpu.make_async_copy(
                ones_v, acc_d.at[dst_v.at[r - 4]], ssod.at[t]).wait()
        pltpu.async_copy(ones_v, acc_s.at[src_v.at[r]], ssot.at[t],
                         add=True)
        pltpu.async_copy(ones_v, acc_d.at[dst_v.at[r]], ssod.at[t],
                         add=True)
        return 0
    lax.fori_loop(0, kc, add, 0)
    for t in range(4, 0, -1):
        m = kc - t + off
        pltpu.make_async_copy(
            ones_v, acc_s.at[src_v.at[m]],
            ssot.at[lax.rem(m - off, 4)]).wait()
        pltpu.make_async_copy(
            ones_v, acc_d.at[dst_v.at[m]],
            ssod.at[lax.rem(m - off, 4)]).wait()

    plsc.subcore_barrier()
    sl = pl.ds(s * ROWS_PER_TILE, ROWS_PER_TILE)
    pltpu.sync_copy(acc_s.at[sl], out_hbm.at[c, 0, sl])
    pltpu.sync_copy(acc_d.at[sl], out_hbm.at[c, 1, sl])


SRC_RING = 4
NROWBUF = 3
GDIST = NROWBUF - 1
SDIST = SRC_RING - 1


def _make_sc_agg(d):
    @functools.partial(
        pl.kernel,
        out_type=jax.ShapeDtypeStruct((NC, N_PAD, d), jnp.float32),
        mesh=_mesh,
        scratch_types=[
            pltpu.VMEM((SRC_RING, CHUNK), jnp.int32),
            pltpu.VMEM((NROWBUF, CHUNK), jnp.int32),
            pltpu.VMEM((NROWBUF, CHUNK, d), jnp.float32),
            pltpu.VMEM_SHARED((ACC_ROWS, d), jnp.float32),
            pltpu.SemaphoreType.DMA((SRC_RING,)),
            pltpu.SemaphoreType.DMA((NROWBUF,)),
            pltpu.SemaphoreType.DMA((NROWBUF,)),
            pltpu.SemaphoreType.DMA((NROWBUF,)),
        ],
    )
    def sc_agg(hs_hbm, src_hbm, dst_hbm, out_hbm, src_v, dst_v, rows_v, acc,
               isems, dsems, gsems, ssems):
        c = lax.axis_index("c")
        s = lax.axis_index("s")
        _zero_vmem_2d(rows_v.at[0], CHUNK, d)
        for base in ZBASES:
            pltpu.sync_copy(
                rows_v.at[0], acc.at[pl.ds(s * ACC_PER_TILE + base, CHUNK)])
        plsc.subcore_barrier()

        off = jnp.where(c == 0, 0, K0)
        kc = jnp.where(c == 0, K0, K1)

        def load_src(j, q):
            pltpu.async_copy(src_hbm.at[s, j + off], src_v.at[q],
                             isems.at[q])

        def wait_src(j, q):
            pltpu.make_async_copy(src_hbm.at[s, j + off], src_v.at[q],
                                  isems.at[q]).wait()

        def load_dst(j, b):
            pltpu.async_copy(dst_hbm.at[s, j + off], dst_v.at[b],
                             dsems.at[b])

        def wait_dst(j, b):
            pltpu.make_async_copy(dst_hbm.at[s, j + off], dst_v.at[b],
                                  dsems.at[b]).wait()

        for k in range(SDIST):
            load_src(k, k)
        for k in range(GDIST):
            load_dst(k, k)
        for m in range(GDIST):
            wait_src(m, m)
            pltpu.async_copy(
                hs_hbm.at[src_v.at[m]], rows_v.at[m], gsems.at[m])

        def step(j, _):
            b = lax.rem(j, NROWBUF)
            pltpu.make_async_copy(
                hs_hbm.at[src_v.at[lax.rem(j, SRC_RING)]], rows_v.at[b],
                gsems.at[b]).wait()
            wait_dst(j, b)
            pltpu.async_copy(
                rows_v.at[b], acc.at[dst_v.at[b]], ssems.at[b], add=True)

            @pl.when(j + GDIST < kc)
            def _():
                bn = lax.rem(j + GDIST, NROWBUF)
                qn = lax.rem(j + GDIST, SRC_RING)
                @pl.when(j >= 1)
                def _():
                    pltpu.make_async_copy(
                        rows_v.at[bn], acc.at[dst_v.at[bn]],
                        ssems.at[bn]).wait()
                load_dst(j + GDIST, bn)
                wait_src(j + GDIST, qn)
                pltpu.async_copy(
                    hs_hbm.at[src_v.at[qn]], rows_v.at[bn], gsems.at[bn])

                @pl.when(j + SDIST < kc)
                def _():
                    load_src(j + SDIST, lax.rem(j + SDIST, SRC_RING))
            return 0
        lax.fori_loop(0, kc, step, 0)

        for t in range(GDIST + 1, 0, -1):
            m = kc - t
            pltpu.make_async_copy(
                rows_v.at[lax.rem(m, NROWBUF)],
                acc.at[dst_v.at[lax.rem(m, NROWBUF)]],
                ssems.at[lax.rem(m, NROWBUF)]).wait()

        plsc.subcore_barrier()
        for base in ZBASES:
            sl = pl.ds(s * ACC_PER_TILE + base, CHUNK)
            pltpu.sync_copy(acc.at[sl], out_hbm.at[c, sl])

    return sc_agg


_sc_agg128 = _make_sc_agg(D_H)


ROW_BLK = 1024
GRID = N_PAD // ROW_BLK


def _norm_from_deg(dref):
    d = dref[:, 0:1] + dref[:, 1:2]
    return jnp.where(d > 0, lax.rsqrt(jnp.maximum(d, 1.0)), 0.0)


def _tc_h_body(x_ref, w_ref, b_ref, h_ref):
    h_ref[...] = jnp.dot(x_ref[...], w_ref[...],
                         preferred_element_type=jnp.float32) + b_ref[...]


def _tc_h(x, w, b):
    return pl.pallas_call(
        _tc_h_body,
        grid=(GRID,),
        in_specs=[
            pl.BlockSpec((ROW_BLK, D_H), lambda i: (i, 0)),
            pl.BlockSpec((D_H, D_H), lambda i: (0, 0)),
            pl.BlockSpec((1, D_H), lambda i: (0, 0)),
        ],
        out_specs=pl.BlockSpec((ROW_BLK, D_H), lambda i: (i, 0)),
        out_shape=jax.ShapeDtypeStruct((N_PAD, D_H), jnp.float32),
    )(x, w, b)


def _tc_norms_body(h_ref, dsrc_ref, ddst_ref, hs0_ref, ns_ref, nd_ref):
    ns = _norm_from_deg(dsrc_ref)
    nd = _norm_from_deg(ddst_ref)
    hs0_ref[...] = h_ref[...] * ns
    ns_ref[...] = ns
    nd_ref[...] = nd


def _tc_norms(h, dsrc, ddst):
    return pl.pallas_call(
        _tc_norms_body,
        grid=(GRID,),
        in_specs=[
            pl.BlockSpec((ROW_BLK, D_H), lambda i: (i, 0)),
            pl.BlockSpec((ROW_BLK, 2), lambda i: (i, 0)),
            pl.BlockSpec((ROW_BLK, 2), lambda i: (i, 0)),
        ],
        out_specs=[
            pl.BlockSpec((ROW_BLK, D_H), lambda i: (i, 0)),
            pl.BlockSpec((ROW_BLK, 1), lambda i: (i, 0)),
            pl.BlockSpec((ROW_BLK, 1), lambda i: (i, 0)),
        ],
        out_shape=[
            jax.ShapeDtypeStruct((N_PAD, D_H), jnp.float32),
            jax.ShapeDtypeStruct((N_PAD, 1), jnp.float32),
            jax.ShapeDtypeStruct((N_PAD, 1), jnp.float32),
        ],
    )(h, dsrc, ddst)


def _tc_layer_body(p_ref, nd_ref, ns_ref, w_ref, b_ref, out_ref):
    agg = (p_ref[0] + p_ref[1]) * nd_ref[...]
    h = jnp.dot(agg, w_ref[...], preferred_element_type=jnp.float32) + b_ref[...]
    h = jnp.maximum(h, 0.0)
    out_ref[...] = h * ns_ref[...]


def _tc_layer(p, nd, ns, w, b):
    return pl.pallas_call(
        _tc_layer_body,
        grid=(GRID,),
        in_specs=[
            pl.BlockSpec((NC, ROW_BLK, D_H), lambda i: (0, i, 0)),
            pl.BlockSpec((ROW_BLK, 1), lambda i: (i, 0)),
            pl.BlockSpec((ROW_BLK, 1), lambda i: (i, 0)),
            pl.BlockSpec((D_H, D_H), lambda i: (0, 0)),
            pl.BlockSpec((1, D_H), lambda i: (0, 0)),
        ],
        out_specs=pl.BlockSpec((ROW_BLK, D_H), lambda i: (i, 0)),
        out_shape=jax.ShapeDtypeStruct((N_PAD, D_H), jnp.float32),
    )(p, nd, ns, w, b)


def _tc_final_body(p_ref, nd_ref, w2_ref, b2_ref, out_ref):
    agg = (p_ref[0] + p_ref[1]) * nd_ref[...]
    out_ref[...] = jnp.dot(agg, w2_ref[...],
                           preferred_element_type=jnp.float32) + b2_ref[...]


def _tc_final(p, nd, w2, b2):
    return pl.pallas_call(
        _tc_final_body,
        grid=(GRID,),
        in_specs=[
            pl.BlockSpec((NC, ROW_BLK, D_H), lambda i: (0, i, 0)),
            pl.BlockSpec((ROW_BLK, 1), lambda i: (i, 0)),
            pl.BlockSpec((D_H, D_OUT), lambda i: (0, 0)),
            pl.BlockSpec((1, D_OUT), lambda i: (0, 0)),
        ],
        out_specs=pl.BlockSpec((ROW_BLK, D_OUT), lambda i: (i, 0)),
        out_shape=jax.ShapeDtypeStruct((N_PAD, D_OUT), jnp.float32),
    )(p, nd, w2, b2)


def kernel(features, edge_index, W_lin, b_lin, W0, b0, W1, b1, W2, b2):
    pad_e = E_PAD - E

    def slabs(v):
        v = jnp.concatenate([v, jnp.full((pad_e,), DUMMY, jnp.int32)])
        return v.reshape(NS, KT, CHUNK)

    src = slabs(edge_index[0])
    dst = slabs(edge_index[1])

    x = jnp.concatenate(
        [features, jnp.zeros((N_PAD - N, features.shape[1]), jnp.float32)])

    degs = _sc_degrees(src, dst)
    dsrc = degs[:, 0, :].T
    ddst = degs[:, 1, :].T

    h = _tc_h(x, W_lin, b_lin.reshape(1, D_H))
    hs0, ns, nd = _tc_norms(h, dsrc, ddst)

    p0 = _sc_agg128(hs0, src, dst)
    hs1 = _tc_layer(p0, nd, ns, W0, b0.reshape(1, D_H))
    p1 = _sc_agg128(hs1, src, dst)
    hs2 = _tc_layer(p1, nd, ns, W1, b1.reshape(1, D_H))
    p2 = _sc_agg128(hs2, src, dst)
    out = _tc_final(p2, nd, W2, b2.reshape(1, D_OUT))
    return out[:N]


# speedup vs baseline: 1.1604x; 1.1604x over previous
import functools

import jax
import jax.numpy as jnp
from jax import lax
from jax.experimental import pallas as pl
from jax.experimental.pallas import tpu as pltpu
from jax.experimental.pallas import tpu_sc as plsc

N = 10000
N_PAD = 10240
ROWS_PER_TILE = 640
ACC_ROWS = 10240
ACC_PER_TILE = 640
DUMMY = N
E = 320000
CHUNK = 120
NC, NS = 2, 16
NW = NC * NS
K0 = 110
K1 = 57
KT = K0 + K1
E_PAD = NS * KT * CHUNK
ZBASES = (0, 120, 240, 360, 480, 520)
D_H = 128
D_OUT = 64

_mesh = plsc.VectorSubcoreMesh(core_axis_name="c", subcore_axis_name="s")


def _zero_vmem_2d(buf, rows, cols):
    def body(r, _):
        for k in range(cols // 16):
            buf[r, pl.ds(16 * k, 16)] = jnp.zeros((16,), jnp.float32)
        return 0
    lax.fori_loop(0, rows, body, 0)


@functools.partial(
    pl.kernel,
    out_type=jax.ShapeDtypeStruct((NC, 2, N_PAD), jnp.float32),
    mesh=_mesh,
    scratch_types=[
        pltpu.VMEM((KT, CHUNK), jnp.int32),
        pltpu.VMEM((KT, CHUNK), jnp.int32),
        pltpu.VMEM((CHUNK,), jnp.float32),
        pltpu.VMEM((ROWS_PER_TILE,), jnp.float32),
        pltpu.VMEM_SHARED((N_PAD,), jnp.float32),
        pltpu.VMEM_SHARED((N_PAD,), jnp.float32),
        pltpu.SemaphoreType.DMA((4,)),
        pltpu.SemaphoreType.DMA((4,)),
    ],
)
def _sc_degrees(src_hbm, dst_hbm, out_hbm, src_v, dst_v, ones_v, zero_v,
                acc_s, acc_d, ssot, ssod):
    c = lax.axis_index("c")
    s = lax.axis_index("s")
    for k in range(CHUNK // 16):
        ones_v[pl.ds(16 * k, 16)] = jnp.ones((16,), jnp.float32)
    if CHUNK % 16:
        ones_v[pl.ds(CHUNK - 16, 16)] = jnp.ones((16,), jnp.float32)
    for k in range(ROWS_PER_TILE // 16):
        zero_v[pl.ds(16 * k, 16)] = jnp.zeros((16,), jnp.float32)
    pltpu.sync_copy(zero_v, acc_s.at[pl.ds(s * ROWS_PER_TILE, ROWS_PER_TILE)])
    pltpu.sync_copy(zero_v, acc_d.at[pl.ds(s * ROWS_PER_TILE, ROWS_PER_TILE)])
    plsc.subcore_barrier()

    pltpu.sync_copy(src_hbm.at[s], src_v)
    pltpu.sync_copy(dst_hbm.at[s], dst_v)
    off = jnp.where(c == 0, 0, K0)
    kc = jnp.where(c == 0, K0, K1)

    def add(j, _):
        t = lax.rem(j, 4)
        r = j + off

        @pl.when(j >= 4)
        def _():
            pltpu.make_async_copy(
                ones_v, acc_s.at[src_v.at[r - 4]], ssot.at[t]).wait()
            pltpu.make_async_copy(
                ones_v, acc_d.at[dst_v.at[r - 4]], ssod.at[t]).wait()
        pltpu.async_copy(ones_v, acc_s.at[src_v.at[r]], ssot.at[t],
                         add=True)
        pltpu.async_copy(ones_v, acc_d.at[dst_v.at[r]], ssod.at[t],
                         add=True)
        return 0
    lax.fori_loop(0, kc, add, 0)
    for t in range(4, 0, -1):
        m = kc - t + off
        pltpu.make_async_copy(
            ones_v, acc_s.at[src_v.at[m]],
            ssot.at[lax.rem(m - off, 4)]).wait()
        pltpu.make_async_copy(
            ones_v, acc_d.at[dst_v.at[m]],
            ssod.at[lax.rem(m - off, 4)]).wait()

    plsc.subcore_barrier()
    sl = pl.ds(s * ROWS_PER_TILE, ROWS_PER_TILE)
    pltpu.sync_copy(acc_s.at[sl], out_hbm.at[c, 0, sl])
    pltpu.sync_copy(acc_d.at[sl], out_hbm.at[c, 1, sl])


IDX_RING = 5
NROWBUF = 3
GDIST = NROWBUF - 1
PDIST = IDX_RING - 1


def _make_sc_agg(d):
    @functools.partial(
        pl.kernel,
        out_type=jax.ShapeDtypeStruct((NC, N_PAD, d), jnp.float32),
        mesh=_mesh,
        scratch_types=[
            pltpu.VMEM((IDX_RING, 2, CHUNK), jnp.int32),
            pltpu.VMEM((NROWBUF, CHUNK, d), jnp.float32),
            pltpu.VMEM_SHARED((ACC_ROWS, d), jnp.float32),
            pltpu.SemaphoreType.DMA((IDX_RING,)),
            pltpu.SemaphoreType.DMA((NROWBUF,)),
            pltpu.SemaphoreType.DMA((NROWBUF,)),
        ],
    )
    def sc_agg(hs_hbm, src_hbm, dst_hbm, out_hbm, idx_v, rows_v, acc,
               isems, gsems, ssems):
        c = lax.axis_index("c")
        s = lax.axis_index("s")
        _zero_vmem_2d(rows_v.at[0], CHUNK, d)
        for base in ZBASES:
            pltpu.sync_copy(
                rows_v.at[0], acc.at[pl.ds(s * ACC_PER_TILE + base, CHUNK)])
        plsc.subcore_barrier()

        off = jnp.where(c == 0, 0, K0)
        kc = jnp.where(c == 0, K0, K1)

        def load_idx(j, q):
            pltpu.async_copy(src_hbm.at[s, j + off], idx_v.at[q, 0],
                             isems.at[q])
            pltpu.async_copy(dst_hbm.at[s, j + off], idx_v.at[q, 1],
                             isems.at[q])

        def wait_idx(j, q):
            pltpu.make_async_copy(src_hbm.at[s, j + off], idx_v.at[q, 0],
                                  isems.at[q]).wait()
            pltpu.make_async_copy(dst_hbm.at[s, j + off], idx_v.at[q, 1],
                                  isems.at[q]).wait()

        for k in range(PDIST):
            load_idx(k, k)
        for m in range(GDIST):
            wait_idx(m, m)
            pltpu.async_copy(
                hs_hbm.at[idx_v.at[m, 0]], rows_v.at[m], gsems.at[m])

        def step(j, _):
            b = lax.rem(j, NROWBUF)
            q = lax.rem(j, IDX_RING)
            pltpu.make_async_copy(
                hs_hbm.at[idx_v.at[q, 0]], rows_v.at[b], gsems.at[b]).wait()
            pltpu.async_copy(
                rows_v.at[b], acc.at[idx_v.at[q, 1]], ssems.at[b], add=True)

            @pl.when(j + GDIST < kc)
            def _():
                bn = lax.rem(j + GDIST, NROWBUF)
                qn = lax.rem(j + GDIST, IDX_RING)
                @pl.when(j >= 1)
                def _():
                    pltpu.make_async_copy(
                        rows_v.at[bn],
                        acc.at[idx_v.at[lax.rem(j - 1, IDX_RING), 1]],
                        ssems.at[bn]).wait()
                wait_idx(j + GDIST, qn)
                pltpu.async_copy(
                    hs_hbm.at[idx_v.at[qn, 0]], rows_v.at[bn], gsems.at[bn])

                @pl.when(j + PDIST < kc)
                def _():
                    qp = lax.rem(j + PDIST, IDX_RING)
                    load_idx(j + PDIST, qp)
            return 0
        lax.fori_loop(0, kc, step, 0)

        for t in range(GDIST + 1, 0, -1):
            m = kc - t
            pltpu.make_async_copy(
                rows_v.at[lax.rem(m, NROWBUF)],
                acc.at[idx_v.at[lax.rem(m, IDX_RING), 1]],
                ssems.at[lax.rem(m, NROWBUF)]).wait()

        plsc.subcore_barrier()
        for base in ZBASES:
            sl = pl.ds(s * ACC_PER_TILE + base, CHUNK)
            pltpu.sync_copy(acc.at[sl], out_hbm.at[c, sl])

    return sc_agg


_sc_agg128 = _make_sc_agg(D_H)


ROW_BLK = 1024
GRID = N_PAD // ROW_BLK


def _norm_from_deg(dref):
    d = dref[:, 0:1] + dref[:, 1:2]
    return jnp.where(d > 0, lax.rsqrt(jnp.maximum(d, 1.0)), 0.0)


def _tc_h_body(x_ref, w_ref, b_ref, h_ref):
    h_ref[...] = jnp.dot(x_ref[...], w_ref[...],
                         preferred_element_type=jnp.float32) + b_ref[...]


def _tc_h(x, w, b):
    return pl.pallas_call(
        _tc_h_body,
        grid=(GRID,),
        in_specs=[
            pl.BlockSpec((ROW_BLK, D_H), lambda i: (i, 0)),
            pl.BlockSpec((D_H, D_H), lambda i: (0, 0)),
            pl.BlockSpec((1, D_H), lambda i: (0, 0)),
        ],
        out_specs=pl.BlockSpec((ROW_BLK, D_H), lambda i: (i, 0)),
        out_shape=jax.ShapeDtypeStruct((N_PAD, D_H), jnp.float32),
    )(x, w, b)


def _tc_norms_body(h_ref, dsrc_ref, ddst_ref, hs0_ref, ns_ref, nd_ref):
    ns = _norm_from_deg(dsrc_ref)
    nd = _norm_from_deg(ddst_ref)
    hs0_ref[...] = h_ref[...] * ns
    ns_ref[...] = ns
    nd_ref[...] = nd


def _tc_norms(h, dsrc, ddst):
    return pl.pallas_call(
        _tc_norms_body,
        grid=(GRID,),
        in_specs=[
            pl.BlockSpec((ROW_BLK, D_H), lambda i: (i, 0)),
            pl.BlockSpec((ROW_BLK, 2), lambda i: (i, 0)),
            pl.BlockSpec((ROW_BLK, 2), lambda i: (i, 0)),
        ],
        out_specs=[
            pl.BlockSpec((ROW_BLK, D_H), lambda i: (i, 0)),
            pl.BlockSpec((ROW_BLK, 1), lambda i: (i, 0)),
            pl.BlockSpec((ROW_BLK, 1), lambda i: (i, 0)),
        ],
        out_shape=[
            jax.ShapeDtypeStruct((N_PAD, D_H), jnp.float32),
            jax.ShapeDtypeStruct((N_PAD, 1), jnp.float32),
            jax.ShapeDtypeStruct((N_PAD, 1), jnp.float32),
        ],
    )(h, dsrc, ddst)


def _tc_layer_body(p_ref, nd_ref, ns_ref, w_ref, b_ref, out_ref):
    agg = (p_ref[0] + p_ref[1]) * nd_ref[...]
    h = jnp.dot(agg, w_ref[...], preferred_element_type=jnp.float32) + b_ref[...]
    h = jnp.maximum(h, 0.0)
    out_ref[...] = h * ns_ref[...]


def _tc_layer(p, nd, ns, w, b):
    return pl.pallas_call(
        _tc_layer_body,
        grid=(GRID,),
        in_specs=[
            pl.BlockSpec((NC, ROW_BLK, D_H), lambda i: (0, i, 0)),
            pl.BlockSpec((ROW_BLK, 1), lambda i: (i, 0)),
            pl.BlockSpec((ROW_BLK, 1), lambda i: (i, 0)),
            pl.BlockSpec((D_H, D_H), lambda i: (0, 0)),
            pl.BlockSpec((1, D_H), lambda i: (0, 0)),
        ],
        out_specs=pl.BlockSpec((ROW_BLK, D_H), lambda i: (i, 0)),
        out_shape=jax.ShapeDtypeStruct((N_PAD, D_H), jnp.float32),
    )(p, nd, ns, w, b)


def _tc_final_body(p_ref, nd_ref, w2_ref, b2_ref, out_ref):
    agg = (p_ref[0] + p_ref[1]) * nd_ref[...]
    out_ref[...] = jnp.dot(agg, w2_ref[...],
                           preferred_element_type=jnp.float32) + b2_ref[...]


def _tc_final(p, nd, w2, b2):
    return pl.pallas_call(
        _tc_final_body,
        grid=(GRID,),
        in_specs=[
            pl.BlockSpec((NC, ROW_BLK, D_H), lambda i: (0, i, 0)),
            pl.BlockSpec((ROW_BLK, 1), lambda i: (i, 0)),
            pl.BlockSpec((D_H, D_OUT), lambda i: (0, 0)),
            pl.BlockSpec((1, D_OUT), lambda i: (0, 0)),
        ],
        out_specs=pl.BlockSpec((ROW_BLK, D_OUT), lambda i: (i, 0)),
        out_shape=jax.ShapeDtypeStruct((N_PAD, D_OUT), jnp.float32),
    )(p, nd, w2, b2)


def kernel(features, edge_index, W_lin, b_lin, W0, b0, W1, b1, W2, b2):
    pad_e = E_PAD - E

    def slabs(v):
        v = jnp.concatenate([v, jnp.full((pad_e,), DUMMY, jnp.int32)])
        return v.reshape(NS, KT, CHUNK)

    src = slabs(edge_index[0])
    dst = slabs(edge_index[1])

    x = jnp.concatenate(
        [features, jnp.zeros((N_PAD - N, features.shape[1]), jnp.float32)])

    degs = _sc_degrees(src, dst)
    dsrc = degs[:, 0, :].T
    ddst = degs[:, 1, :].T

    h = _tc_h(x, W_lin, b_lin.reshape(1, D_H))
    hs0, ns, nd = _tc_norms(h, dsrc, ddst)

    p0 = _sc_agg128(hs0, src, dst)
    hs1 = _tc_layer(p0, nd, ns, W0, b0.reshape(1, D_H))
    p1 = _sc_agg128(hs1, src, dst)
    hs2 = _tc_layer(p1, nd, ns, W1, b1.reshape(1, D_H))
    p2 = _sc_agg128(hs2, src, dst)
    out = _tc_final(p2, nd, W2, b2.reshape(1, D_OUT))
    return out[:N]


# speedup vs baseline: 1.1784x; 1.0155x over previous
import functools

import jax
import jax.numpy as jnp
from jax import lax
from jax.experimental import pallas as pl
from jax.experimental.pallas import tpu as pltpu
from jax.experimental.pallas import tpu_sc as plsc

N = 10000
N_PAD = 10240
ROWS_PER_TILE = 640
ACC_ROWS = 10240
ACC_PER_TILE = 640
DUMMY = N
E = 320000
CHUNK = 120
NC, NS = 2, 16
NW = NC * NS
K0 = 107
K1 = 60
KT = K0 + K1
E_PAD = NS * KT * CHUNK
ZBASES = (0, 120, 240, 360, 480, 520)
D_H = 128
D_OUT = 64

_mesh = plsc.VectorSubcoreMesh(core_axis_name="c", subcore_axis_name="s")


def _zero_vmem_2d(buf, rows, cols):
    def body(r, _):
        for k in range(cols // 16):
            buf[r, pl.ds(16 * k, 16)] = jnp.zeros((16,), jnp.float32)
        return 0
    lax.fori_loop(0, rows, body, 0)


@functools.partial(
    pl.kernel,
    out_type=jax.ShapeDtypeStruct((NC, 2, N_PAD), jnp.float32),
    mesh=_mesh,
    scratch_types=[
        pltpu.VMEM((KT, CHUNK), jnp.int32),
        pltpu.VMEM((KT, CHUNK), jnp.int32),
        pltpu.VMEM((CHUNK,), jnp.float32),
        pltpu.VMEM((ROWS_PER_TILE,), jnp.float32),
        pltpu.VMEM_SHARED((N_PAD,), jnp.float32),
        pltpu.VMEM_SHARED((N_PAD,), jnp.float32),
        pltpu.SemaphoreType.DMA((4,)),
        pltpu.SemaphoreType.DMA((4,)),
    ],
)
def _sc_degrees(src_hbm, dst_hbm, out_hbm, src_v, dst_v, ones_v, zero_v,
                acc_s, acc_d, ssot, ssod):
    c = lax.axis_index("c")
    s = lax.axis_index("s")
    for k in range(CHUNK // 16):
        ones_v[pl.ds(16 * k, 16)] = jnp.ones((16,), jnp.float32)
    if CHUNK % 16:
        ones_v[pl.ds(CHUNK - 16, 16)] = jnp.ones((16,), jnp.float32)
    for k in range(ROWS_PER_TILE // 16):
        zero_v[pl.ds(16 * k, 16)] = jnp.zeros((16,), jnp.float32)
    pltpu.sync_copy(zero_v, acc_s.at[pl.ds(s * ROWS_PER_TILE, ROWS_PER_TILE)])
    pltpu.sync_copy(zero_v, acc_d.at[pl.ds(s * ROWS_PER_TILE, ROWS_PER_TILE)])
    plsc.subcore_barrier()

    pltpu.sync_copy(src_hbm.at[s], src_v)
    pltpu.sync_copy(dst_hbm.at[s], dst_v)
    off = jnp.where(c == 0, 0, K0)
    kc = jnp.where(c == 0, K0, K1)

    def add(j, _):
        t = lax.rem(j, 4)
        r = j + off

        @pl.when(j >= 4)
        def _():
            pltpu.make_async_copy(
                ones_v, acc_s.at[src_v.at[r - 4]], ssot.at[t]).wait()
            pltpu.make_async_copy(
                ones_v, acc_d.at[dst_v.at[r - 4]], ssod.at[t]).wait()
        pltpu.async_copy(ones_v, acc_s.at[src_v.at[r]], ssot.at[t],
                         add=True)
        pltpu.async_copy(ones_v, acc_d.at[dst_v.at[r]], ssod.at[t],
                         add=True)
        return 0
    lax.fori_loop(0, kc, add, 0)
    for t in range(4, 0, -1):
        m = kc - t + off
        pltpu.make_async_copy(
            ones_v, acc_s.at[src_v.at[m]],
            ssot.at[lax.rem(m - off, 4)]).wait()
        pltpu.make_async_copy(
            ones_v, acc_d.at[dst_v.at[m]],
            ssod.at[lax.rem(m - off, 4)]).wait()

    plsc.subcore_barrier()
    sl = pl.ds(s * ROWS_PER_TILE, ROWS_PER_TILE)
    pltpu.sync_copy(acc_s.at[sl], out_hbm.at[c, 0, sl])
    pltpu.sync_copy(acc_d.at[sl], out_hbm.at[c, 1, sl])


IDX_RING = 5
NROWBUF = 3
GDIST = NROWBUF - 1
PDIST = IDX_RING - 1


def _make_sc_agg(d):
    @functools.partial(
        pl.kernel,
        out_type=jax.ShapeDtypeStruct((NC, N_PAD, d), jnp.float32),
        mesh=_mesh,
        scratch_types=[
            pltpu.VMEM((IDX_RING, 2, CHUNK), jnp.int32),
            pltpu.VMEM((NROWBUF, CHUNK, d), jnp.float32),
            pltpu.VMEM_SHARED((ACC_ROWS, d), jnp.float32),
            pltpu.SemaphoreType.DMA((IDX_RING,)),
            pltpu.SemaphoreType.DMA((NROWBUF,)),
            pltpu.SemaphoreType.DMA((NROWBUF,)),
        ],
    )
    def sc_agg(hs_hbm, src_hbm, dst_hbm, out_hbm, idx_v, rows_v, acc,
               isems, gsems, ssems):
        c = lax.axis_index("c")
        s = lax.axis_index("s")
        _zero_vmem_2d(rows_v.at[0], CHUNK, d)
        for base in ZBASES:
            pltpu.sync_copy(
                rows_v.at[0], acc.at[pl.ds(s * ACC_PER_TILE + base, CHUNK)])
        plsc.subcore_barrier()

        off = jnp.where(c == 0, 0, K0)
        kc = jnp.where(c == 0, K0, K1)

        def load_idx(j, q):
            pltpu.async_copy(src_hbm.at[s, j + off], idx_v.at[q, 0],
                             isems.at[q])
            pltpu.async_copy(dst_hbm.at[s, j + off], idx_v.at[q, 1],
                             isems.at[q])

        def wait_idx(j, q):
            pltpu.make_async_copy(src_hbm.at[s, j + off], idx_v.at[q, 0],
                                  isems.at[q]).wait()
            pltpu.make_async_copy(dst_hbm.at[s, j + off], idx_v.at[q, 1],
                                  isems.at[q]).wait()

        for k in range(PDIST):
            load_idx(k, k)
        for m in range(GDIST):
            wait_idx(m, m)
            pltpu.async_copy(
                hs_hbm.at[idx_v.at[m, 0]], rows_v.at[m], gsems.at[m])

        def step(j, _):
            b = lax.rem(j, NROWBUF)
            q = lax.rem(j, IDX_RING)
            pltpu.make_async_copy(
                hs_hbm.at[idx_v.at[q, 0]], rows_v.at[b], gsems.at[b]).wait()
            pltpu.async_copy(
                rows_v.at[b], acc.at[idx_v.at[q, 1]], ssems.at[b], add=True)

            @pl.when(j + GDIST < kc)
            def _():
                bn = lax.rem(j + GDIST, NROWBUF)
                qn = lax.rem(j + GDIST, IDX_RING)
                @pl.when(j >= 1)
                def _():
                    pltpu.make_async_copy(
                        rows_v.at[bn],
                        acc.at[idx_v.at[lax.rem(j - 1, IDX_RING), 1]],
                        ssems.at[bn]).wait()
                wait_idx(j + GDIST, qn)
                pltpu.async_copy(
                    hs_hbm.at[idx_v.at[qn, 0]], rows_v.at[bn], gsems.at[bn])

                @pl.when(j + PDIST < kc)
                def _():
                    qp = lax.rem(j + PDIST, IDX_RING)
                    load_idx(j + PDIST, qp)
            return 0
        lax.fori_loop(0, kc, step, 0)

        for t in range(GDIST + 1, 0, -1):
            m = kc - t
            pltpu.make_async_copy(
                rows_v.at[lax.rem(m, NROWBUF)],
                acc.at[idx_v.at[lax.rem(m, IDX_RING), 1]],
                ssems.at[lax.rem(m, NROWBUF)]).wait()

        plsc.subcore_barrier()
        for base in ZBASES:
            sl = pl.ds(s * ACC_PER_TILE + base, CHUNK)
            pltpu.sync_copy(acc.at[sl], out_hbm.at[c, sl])

    return sc_agg


_sc_agg128 = _make_sc_agg(D_H)


ROW_BLK = 1024
GRID = N_PAD // ROW_BLK


def _norm_from_deg(dref):
    d = dref[:, 0:1] + dref[:, 1:2]
    return jnp.where(d > 0, lax.rsqrt(jnp.maximum(d, 1.0)), 0.0)


def _tc_h_body(x_ref, w_ref, b_ref, h_ref):
    h_ref[...] = jnp.dot(x_ref[...], w_ref[...],
                         preferred_element_type=jnp.float32) + b_ref[...]


def _tc_h(x, w, b):
    return pl.pallas_call(
        _tc_h_body,
        grid=(GRID,),
        in_specs=[
            pl.BlockSpec((ROW_BLK, D_H), lambda i: (i, 0)),
            pl.BlockSpec((D_H, D_H), lambda i: (0, 0)),
            pl.BlockSpec((1, D_H), lambda i: (0, 0)),
        ],
        out_specs=pl.BlockSpec((ROW_BLK, D_H), lambda i: (i, 0)),
        out_shape=jax.ShapeDtypeStruct((N_PAD, D_H), jnp.float32),
    )(x, w, b)


def _tc_norms_body(h_ref, dsrc_ref, ddst_ref, hs0_ref, ns_ref, nd_ref):
    ns = _norm_from_deg(dsrc_ref)
    nd = _norm_from_deg(ddst_ref)
    hs0_ref[...] = h_ref[...] * ns
    ns_ref[...] = ns
    nd_ref[...] = nd


def _tc_norms(h, dsrc, ddst):
    return pl.pallas_call(
        _tc_norms_body,
        grid=(GRID,),
        in_specs=[
            pl.BlockSpec((ROW_BLK, D_H), lambda i: (i, 0)),
            pl.BlockSpec((ROW_BLK, 2), lambda i: (i, 0)),
            pl.BlockSpec((ROW_BLK, 2), lambda i: (i, 0)),
        ],
        out_specs=[
            pl.BlockSpec((ROW_BLK, D_H), lambda i: (i, 0)),
            pl.BlockSpec((ROW_BLK, 1), lambda i: (i, 0)),
            pl.BlockSpec((ROW_BLK, 1), lambda i: (i, 0)),
        ],
        out_shape=[
            jax.ShapeDtypeStruct((N_PAD, D_H), jnp.float32),
            jax.ShapeDtypeStruct((N_PAD, 1), jnp.float32),
            jax.ShapeDtypeStruct((N_PAD, 1), jnp.float32),
        ],
    )(h, dsrc, ddst)


def _tc_layer_body(p_ref, nd_ref, ns_ref, w_ref, b_ref, out_ref):
    agg = (p_ref[0] + p_ref[1]) * nd_ref[...]
    h = jnp.dot(agg, w_ref[...], preferred_element_type=jnp.float32) + b_ref[...]
    h = jnp.maximum(h, 0.0)
    out_ref[...] = h * ns_ref[...]


def _tc_layer(p, nd, ns, w, b):
    return pl.pallas_call(
        _tc_layer_body,
        grid=(GRID,),
        in_specs=[
            pl.BlockSpec((NC, ROW_BLK, D_H), lambda i: (0, i, 0)),
            pl.BlockSpec((ROW_BLK, 1), lambda i: (i, 0)),
            pl.BlockSpec((ROW_BLK, 1), lambda i: (i, 0)),
            pl.BlockSpec((D_H, D_H), lambda i: (0, 0)),
            pl.BlockSpec((1, D_H), lambda i: (0, 0)),
        ],
        out_specs=pl.BlockSpec((ROW_BLK, D_H), lambda i: (i, 0)),
        out_shape=jax.ShapeDtypeStruct((N_PAD, D_H), jnp.float32),
    )(p, nd, ns, w, b)


def _tc_final_body(p_ref, nd_ref, w2_ref, b2_ref, out_ref):
    agg = (p_ref[0] + p_ref[1]) * nd_ref[...]
    out_ref[...] = jnp.dot(agg, w2_ref[...],
                           preferred_element_type=jnp.float32) + b2_ref[...]


def _tc_final(p, nd, w2, b2):
    return pl.pallas_call(
        _tc_final_body,
        grid=(GRID,),
        in_specs=[
            pl.BlockSpec((NC, ROW_BLK, D_H), lambda i: (0, i, 0)),
            pl.BlockSpec((ROW_BLK, 1), lambda i: (i, 0)),
            pl.BlockSpec((D_H, D_OUT), lambda i: (0, 0)),
            pl.BlockSpec((1, D_OUT), lambda i: (0, 0)),
        ],
        out_specs=pl.BlockSpec((ROW_BLK, D_OUT), lambda i: (i, 0)),
        out_shape=jax.ShapeDtypeStruct((N_PAD, D_OUT), jnp.float32),
    )(p, nd, w2, b2)


def kernel(features, edge_index, W_lin, b_lin, W0, b0, W1, b1, W2, b2):
    pad_e = E_PAD - E

    def slabs(v):
        v = jnp.concatenate([v, jnp.full((pad_e,), DUMMY, jnp.int32)])
        return v.reshape(NS, KT, CHUNK)

    src = slabs(edge_index[0])
    dst = slabs(edge_index[1])

    x = jnp.concatenate(
        [features, jnp.zeros((N_PAD - N, features.shape[1]), jnp.float32)])

    degs = _sc_degrees(src, dst)
    dsrc = degs[:, 0, :].T
    ddst = degs[:, 1, :].T

    h = _tc_h(x, W_lin, b_lin.reshape(1, D_H))
    hs0, ns, nd = _tc_norms(h, dsrc, ddst)

    p0 = _sc_agg128(hs0, src, dst)
    hs1 = _tc_layer(p0, nd, ns, W0, b0.reshape(1, D_H))
    p1 = _sc_agg128(hs1, src, dst)
    hs2 = _tc_layer(p1, nd, ns, W1, b1.reshape(1, D_H))
    p2 = _sc_agg128(hs2, src, dst)
    out = _tc_final(p2, nd, W2, b2.reshape(1, D_OUT))
    return out[:N]


# speedup vs baseline: 1.2046x; 1.0222x over previous
import functools

import jax
import jax.numpy as jnp
from jax import lax
from jax.experimental import pallas as pl
from jax.experimental.pallas import tpu as pltpu
from jax.experimental.pallas import tpu_sc as plsc

N = 10000
N_PAD = 10240
ROWS_PER_TILE = 640
ACC_ROWS = 10240
ACC_PER_TILE = 640
DUMMY = N
E = 320000
CHUNK = 120
NC, NS = 2, 16
NW = NC * NS
K0 = 104
K1 = 63
KT = K0 + K1
E_PAD = NS * KT * CHUNK
ZBASES = (0, 120, 240, 360, 480, 520)
D_H = 128
D_OUT = 64

_mesh = plsc.VectorSubcoreMesh(core_axis_name="c", subcore_axis_name="s")


def _zero_vmem_2d(buf, rows, cols):
    def body(r, _):
        for k in range(cols // 16):
            buf[r, pl.ds(16 * k, 16)] = jnp.zeros((16,), jnp.float32)
        return 0
    lax.fori_loop(0, rows, body, 0)


@functools.partial(
    pl.kernel,
    out_type=jax.ShapeDtypeStruct((NC, 2, N_PAD), jnp.float32),
    mesh=_mesh,
    scratch_types=[
        pltpu.VMEM((KT, CHUNK), jnp.int32),
        pltpu.VMEM((KT, CHUNK), jnp.int32),
        pltpu.VMEM((CHUNK,), jnp.float32),
        pltpu.VMEM((ROWS_PER_TILE,), jnp.float32),
        pltpu.VMEM_SHARED((N_PAD,), jnp.float32),
        pltpu.VMEM_SHARED((N_PAD,), jnp.float32),
        pltpu.SemaphoreType.DMA((4,)),
        pltpu.SemaphoreType.DMA((4,)),
    ],
)
def _sc_degrees(src_hbm, dst_hbm, out_hbm, src_v, dst_v, ones_v, zero_v,
                acc_s, acc_d, ssot, ssod):
    c = lax.axis_index("c")
    s = lax.axis_index("s")
    for k in range(CHUNK // 16):
        ones_v[pl.ds(16 * k, 16)] = jnp.ones((16,), jnp.float32)
    if CHUNK % 16:
        ones_v[pl.ds(CHUNK - 16, 16)] = jnp.ones((16,), jnp.float32)
    for k in range(ROWS_PER_TILE // 16):
        zero_v[pl.ds(16 * k, 16)] = jnp.zeros((16,), jnp.float32)
    pltpu.sync_copy(zero_v, acc_s.at[pl.ds(s * ROWS_PER_TILE, ROWS_PER_TILE)])
    pltpu.sync_copy(zero_v, acc_d.at[pl.ds(s * ROWS_PER_TILE, ROWS_PER_TILE)])
    plsc.subcore_barrier()

    pltpu.sync_copy(src_hbm.at[s], src_v)
    pltpu.sync_copy(dst_hbm.at[s], dst_v)
    off = jnp.where(c == 0, 0, K0)
    kc = jnp.where(c == 0, K0, K1)

    def add(j, _):
        t = lax.rem(j, 4)
        r = j + off

        @pl.when(j >= 4)
        def _():
            pltpu.make_async_copy(
                ones_v, acc_s.at[src_v.at[r - 4]], ssot.at[t]).wait()
            pltpu.make_async_copy(
                ones_v, acc_d.at[dst_v.at[r - 4]], ssod.at[t]).wait()
        pltpu.async_copy(ones_v, acc_s.at[src_v.at[r]], ssot.at[t],
                         add=True)
        pltpu.async_copy(ones_v, acc_d.at[dst_v.at[r]], ssod.at[t],
                         add=True)
        return 0
    lax.fori_loop(0, kc, add, 0)
    for t in range(4, 0, -1):
        m = kc - t + off
        pltpu.make_async_copy(
            ones_v, acc_s.at[src_v.at[m]],
            ssot.at[lax.rem(m - off, 4)]).wait()
        pltpu.make_async_copy(
            ones_v, acc_d.at[dst_v.at[m]],
            ssod.at[lax.rem(m - off, 4)]).wait()

    plsc.subcore_barrier()
    sl = pl.ds(s * ROWS_PER_TILE, ROWS_PER_TILE)
    pltpu.sync_copy(acc_s.at[sl], out_hbm.at[c, 0, sl])
    pltpu.sync_copy(acc_d.at[sl], out_hbm.at[c, 1, sl])


IDX_RING = 5
NROWBUF = 3
GDIST = NROWBUF - 1
PDIST = IDX_RING - 1


def _make_sc_agg(d):
    @functools.partial(
        pl.kernel,
        out_type=jax.ShapeDtypeStruct((NC, N_PAD, d), jnp.float32),
        mesh=_mesh,
        scratch_types=[
            pltpu.VMEM((IDX_RING, 2, CHUNK), jnp.int32),
            pltpu.VMEM((NROWBUF, CHUNK, d), jnp.float32),
            pltpu.VMEM_SHARED((ACC_ROWS, d), jnp.float32),
            pltpu.SemaphoreType.DMA((IDX_RING,)),
            pltpu.SemaphoreType.DMA((NROWBUF,)),
            pltpu.SemaphoreType.DMA((NROWBUF,)),
        ],
    )
    def sc_agg(hs_hbm, src_hbm, dst_hbm, out_hbm, idx_v, rows_v, acc,
               isems, gsems, ssems):
        c = lax.axis_index("c")
        s = lax.axis_index("s")
        _zero_vmem_2d(rows_v.at[0], CHUNK, d)
        for base in ZBASES:
            pltpu.sync_copy(
                rows_v.at[0], acc.at[pl.ds(s * ACC_PER_TILE + base, CHUNK)])
        plsc.subcore_barrier()

        off = jnp.where(c == 0, 0, K0)
        kc = jnp.where(c == 0, K0, K1)

        def load_idx(j, q):
            pltpu.async_copy(src_hbm.at[s, j + off], idx_v.at[q, 0],
                             isems.at[q])
            pltpu.async_copy(dst_hbm.at[s, j + off], idx_v.at[q, 1],
                             isems.at[q])

        def wait_idx(j, q):
            pltpu.make_async_copy(src_hbm.at[s, j + off], idx_v.at[q, 0],
                                  isems.at[q]).wait()
            pltpu.make_async_copy(dst_hbm.at[s, j + off], idx_v.at[q, 1],
                                  isems.at[q]).wait()

        for k in range(PDIST):
            load_idx(k, k)
        for m in range(GDIST):
            wait_idx(m, m)
            pltpu.async_copy(
                hs_hbm.at[idx_v.at[m, 0]], rows_v.at[m], gsems.at[m])

        def step(j, _):
            b = lax.rem(j, NROWBUF)
            q = lax.rem(j, IDX_RING)
            pltpu.make_async_copy(
                hs_hbm.at[idx_v.at[q, 0]], rows_v.at[b], gsems.at[b]).wait()
            pltpu.async_copy(
                rows_v.at[b], acc.at[idx_v.at[q, 1]], ssems.at[b], add=True)

            @pl.when(j + GDIST < kc)
            def _():
                bn = lax.rem(j + GDIST, NROWBUF)
                qn = lax.rem(j + GDIST, IDX_RING)
                @pl.when(j >= 1)
                def _():
                    pltpu.make_async_copy(
                        rows_v.at[bn],
                        acc.at[idx_v.at[lax.rem(j - 1, IDX_RING), 1]],
                        ssems.at[bn]).wait()
                wait_idx(j + GDIST, qn)
                pltpu.async_copy(
                    hs_hbm.at[idx_v.at[qn, 0]], rows_v.at[bn], gsems.at[bn])

                @pl.when(j + PDIST < kc)
                def _():
                    qp = lax.rem(j + PDIST, IDX_RING)
                    load_idx(j + PDIST, qp)
            return 0
        lax.fori_loop(0, kc, step, 0)

        for t in range(GDIST + 1, 0, -1):
            m = kc - t
            pltpu.make_async_copy(
                rows_v.at[lax.rem(m, NROWBUF)],
                acc.at[idx_v.at[lax.rem(m, IDX_RING), 1]],
                ssems.at[lax.rem(m, NROWBUF)]).wait()

        plsc.subcore_barrier()
        for base in ZBASES:
            sl = pl.ds(s * ACC_PER_TILE + base, CHUNK)
            pltpu.sync_copy(acc.at[sl], out_hbm.at[c, sl])

    return sc_agg


_sc_agg128 = _make_sc_agg(D_H)


ROW_BLK = 1024
GRID = N_PAD // ROW_BLK


def _norm_from_deg(dref):
    d = dref[:, 0:1] + dref[:, 1:2]
    return jnp.where(d > 0, lax.rsqrt(jnp.maximum(d, 1.0)), 0.0)


def _tc_h_body(x_ref, w_ref, b_ref, h_ref):
    h_ref[...] = jnp.dot(x_ref[...], w_ref[...],
                         preferred_element_type=jnp.float32) + b_ref[...]


def _tc_h(x, w, b):
    return pl.pallas_call(
        _tc_h_body,
        grid=(GRID,),
        in_specs=[
            pl.BlockSpec((ROW_BLK, D_H), lambda i: (i, 0)),
            pl.BlockSpec((D_H, D_H), lambda i: (0, 0)),
            pl.BlockSpec((1, D_H), lambda i: (0, 0)),
        ],
        out_specs=pl.BlockSpec((ROW_BLK, D_H), lambda i: (i, 0)),
        out_shape=jax.ShapeDtypeStruct((N_PAD, D_H), jnp.float32),
    )(x, w, b)


def _tc_norms_body(h_ref, dsrc_ref, ddst_ref, hs0_ref, ns_ref, nd_ref):
    ns = _norm_from_deg(dsrc_ref)
    nd = _norm_from_deg(ddst_ref)
    hs0_ref[...] = h_ref[...] * ns
    ns_ref[...] = ns
    nd_ref[...] = nd


def _tc_norms(h, dsrc, ddst):
    return pl.pallas_call(
        _tc_norms_body,
        grid=(GRID,),
        in_specs=[
            pl.BlockSpec((ROW_BLK, D_H), lambda i: (i, 0)),
            pl.BlockSpec((ROW_BLK, 2), lambda i: (i, 0)),
            pl.BlockSpec((ROW_BLK, 2), lambda i: (i, 0)),
        ],
        out_specs=[
            pl.BlockSpec((ROW_BLK, D_H), lambda i: (i, 0)),
            pl.BlockSpec((ROW_BLK, 1), lambda i: (i, 0)),
            pl.BlockSpec((ROW_BLK, 1), lambda i: (i, 0)),
        ],
        out_shape=[
            jax.ShapeDtypeStruct((N_PAD, D_H), jnp.float32),
            jax.ShapeDtypeStruct((N_PAD, 1), jnp.float32),
            jax.ShapeDtypeStruct((N_PAD, 1), jnp.float32),
        ],
    )(h, dsrc, ddst)


def _tc_layer_body(p_ref, nd_ref, ns_ref, w_ref, b_ref, out_ref):
    agg = (p_ref[0] + p_ref[1]) * nd_ref[...]
    h = jnp.dot(agg, w_ref[...], preferred_element_type=jnp.float32) + b_ref[...]
    h = jnp.maximum(h, 0.0)
    out_ref[...] = h * ns_ref[...]


def _tc_layer(p, nd, ns, w, b):
    return pl.pallas_call(
        _tc_layer_body,
        grid=(GRID,),
        in_specs=[
            pl.BlockSpec((NC, ROW_BLK, D_H), lambda i: (0, i, 0)),
            pl.BlockSpec((ROW_BLK, 1), lambda i: (i, 0)),
            pl.BlockSpec((ROW_BLK, 1), lambda i: (i, 0)),
            pl.BlockSpec((D_H, D_H), lambda i: (0, 0)),
            pl.BlockSpec((1, D_H), lambda i: (0, 0)),
        ],
        out_specs=pl.BlockSpec((ROW_BLK, D_H), lambda i: (i, 0)),
        out_shape=jax.ShapeDtypeStruct((N_PAD, D_H), jnp.float32),
    )(p, nd, ns, w, b)


def _tc_final_body(p_ref, nd_ref, w2_ref, b2_ref, out_ref):
    agg = (p_ref[0] + p_ref[1]) * nd_ref[...]
    out_ref[...] = jnp.dot(agg, w2_ref[...],
                           preferred_element_type=jnp.float32) + b2_ref[...]


def _tc_final(p, nd, w2, b2):
    return pl.pallas_call(
        _tc_final_body,
        grid=(GRID,),
        in_specs=[
            pl.BlockSpec((NC, ROW_BLK, D_H), lambda i: (0, i, 0)),
            pl.BlockSpec((ROW_BLK, 1), lambda i: (i, 0)),
            pl.BlockSpec((D_H, D_OUT), lambda i: (0, 0)),
            pl.BlockSpec((1, D_OUT), lambda i: (0, 0)),
        ],
        out_specs=pl.BlockSpec((ROW_BLK, D_OUT), lambda i: (i, 0)),
        out_shape=jax.ShapeDtypeStruct((N_PAD, D_OUT), jnp.float32),
    )(p, nd, w2, b2)


def kernel(features, edge_index, W_lin, b_lin, W0, b0, W1, b1, W2, b2):
    pad_e = E_PAD - E

    def slabs(v):
        v = jnp.concatenate([v, jnp.full((pad_e,), DUMMY, jnp.int32)])
        return v.reshape(NS, KT, CHUNK)

    src = slabs(edge_index[0])
    dst = slabs(edge_index[1])

    x = jnp.concatenate(
        [features, jnp.zeros((N_PAD - N, features.shape[1]), jnp.float32)])

    degs = _sc_degrees(src, dst)
    dsrc = degs[:, 0, :].T
    ddst = degs[:, 1, :].T

    h = _tc_h(x, W_lin, b_lin.reshape(1, D_H))
    hs0, ns, nd = _tc_norms(h, dsrc, ddst)

    p0 = _sc_agg128(hs0, src, dst)
    hs1 = _tc_layer(p0, nd, ns, W0, b0.reshape(1, D_H))
    p1 = _sc_agg128(hs1, src, dst)
    hs2 = _tc_layer(p1, nd, ns, W1, b1.reshape(1, D_H))
    p2 = _sc_agg128(hs2, src, dst)
    out = _tc_final(p2, nd, W2, b2.reshape(1, D_OUT))
    return out[:N]


# speedup vs baseline: 1.2222x; 1.0146x over previous
import functools

import jax
import jax.numpy as jnp
from jax import lax
from jax.experimental import pallas as pl
from jax.experimental.pallas import tpu as pltpu
from jax.experimental.pallas import tpu_sc as plsc

N = 10000
N_PAD = 10240
ROWS_PER_TILE = 640
ACC_ROWS = 10240
ACC_PER_TILE = 640
DUMMY = N
E = 320000
CHUNK = 120
NC, NS = 2, 16
NW = NC * NS
K0 = 101
K1 = 66
KT = K0 + K1
E_PAD = NS * KT * CHUNK
ZBASES = (0, 120, 240, 360, 480, 520)
D_H = 128
D_OUT = 64

_mesh = plsc.VectorSubcoreMesh(core_axis_name="c", subcore_axis_name="s")


def _zero_vmem_2d(buf, rows, cols):
    def body(r, _):
        for k in range(cols // 16):
            buf[r, pl.ds(16 * k, 16)] = jnp.zeros((16,), jnp.float32)
        return 0
    lax.fori_loop(0, rows, body, 0)


@functools.partial(
    pl.kernel,
    out_type=jax.ShapeDtypeStruct((NC, 2, N_PAD), jnp.float32),
    mesh=_mesh,
    scratch_types=[
        pltpu.VMEM((KT, CHUNK), jnp.int32),
        pltpu.VMEM((KT, CHUNK), jnp.int32),
        pltpu.VMEM((CHUNK,), jnp.float32),
        pltpu.VMEM((ROWS_PER_TILE,), jnp.float32),
        pltpu.VMEM_SHARED((N_PAD,), jnp.float32),
        pltpu.VMEM_SHARED((N_PAD,), jnp.float32),
        pltpu.SemaphoreType.DMA((4,)),
        pltpu.SemaphoreType.DMA((4,)),
    ],
)
def _sc_degrees(src_hbm, dst_hbm, out_hbm, src_v, dst_v, ones_v, zero_v,
                acc_s, acc_d, ssot, ssod):
    c = lax.axis_index("c")
    s = lax.axis_index("s")
    for k in range(CHUNK // 16):
        ones_v[pl.ds(16 * k, 16)] = jnp.ones((16,), jnp.float32)
    if CHUNK % 16:
        ones_v[pl.ds(CHUNK - 16, 16)] = jnp.ones((16,), jnp.float32)
    for k in range(ROWS_PER_TILE // 16):
        zero_v[pl.ds(16 * k, 16)] = jnp.zeros((16,), jnp.float32)
    pltpu.sync_copy(zero_v, acc_s.at[pl.ds(s * ROWS_PER_TILE, ROWS_PER_TILE)])
    pltpu.sync_copy(zero_v, acc_d.at[pl.ds(s * ROWS_PER_TILE, ROWS_PER_TILE)])
    plsc.subcore_barrier()

    pltpu.sync_copy(src_hbm.at[s], src_v)
    pltpu.sync_copy(dst_hbm.at[s], dst_v)
    off = jnp.where(c == 0, 0, K0)
    kc = jnp.where(c == 0, K0, K1)

    def add(j, _):
        t = lax.rem(j, 4)
        r = j + off

        @pl.when(j >= 4)
        def _():
            pltpu.make_async_copy(
                ones_v, acc_s.at[src_v.at[r - 4]], ssot.at[t]).wait()
            pltpu.make_async_copy(
                ones_v, acc_d.at[dst_v.at[r - 4]], ssod.at[t]).wait()
        pltpu.async_copy(ones_v, acc_s.at[src_v.at[r]], ssot.at[t],
                         add=True)
        pltpu.async_copy(ones_v, acc_d.at[dst_v.at[r]], ssod.at[t],
                         add=True)
        return 0
    lax.fori_loop(0, kc, add, 0)
    for t in range(4, 0, -1):
        m = kc - t + off
        pltpu.make_async_copy(
            ones_v, acc_s.at[src_v.at[m]],
            ssot.at[lax.rem(m - off, 4)]).wait()
        pltpu.make_async_copy(
            ones_v, acc_d.at[dst_v.at[m]],
            ssod.at[lax.rem(m - off, 4)]).wait()

    plsc.subcore_barrier()
    sl = pl.ds(s * ROWS_PER_TILE, ROWS_PER_TILE)
    pltpu.sync_copy(acc_s.at[sl], out_hbm.at[c, 0, sl])
    pltpu.sync_copy(acc_d.at[sl], out_hbm.at[c, 1, sl])


IDX_RING = 5
NROWBUF = 3
GDIST = NROWBUF - 1
PDIST = IDX_RING - 1


def _make_sc_agg(d):
    @functools.partial(
        pl.kernel,
        out_type=jax.ShapeDtypeStruct((NC, N_PAD, d), jnp.float32),
        mesh=_mesh,
        scratch_types=[
            pltpu.VMEM((IDX_RING, 2, CHUNK), jnp.int32),
            pltpu.VMEM((NROWBUF, CHUNK, d), jnp.float32),
            pltpu.VMEM_SHARED((ACC_ROWS, d), jnp.float32),
            pltpu.SemaphoreType.DMA((IDX_RING,)),
            pltpu.SemaphoreType.DMA((NROWBUF,)),
            pltpu.SemaphoreType.DMA((NROWBUF,)),
        ],
    )
    def sc_agg(hs_hbm, src_hbm, dst_hbm, out_hbm, idx_v, rows_v, acc,
               isems, gsems, ssems):
        c = lax.axis_index("c")
        s = lax.axis_index("s")
        _zero_vmem_2d(rows_v.at[0], CHUNK, d)
        for base in ZBASES:
            pltpu.sync_copy(
                rows_v.at[0], acc.at[pl.ds(s * ACC_PER_TILE + base, CHUNK)])
        plsc.subcore_barrier()

        off = jnp.where(c == 0, 0, K0)
        kc = jnp.where(c == 0, K0, K1)

        def load_idx(j, q):
            pltpu.async_copy(src_hbm.at[s, j + off], idx_v.at[q, 0],
                             isems.at[q])
            pltpu.async_copy(dst_hbm.at[s, j + off], idx_v.at[q, 1],
                             isems.at[q])

        def wait_idx(j, q):
            pltpu.make_async_copy(src_hbm.at[s, j + off], idx_v.at[q, 0],
                                  isems.at[q]).wait()
            pltpu.make_async_copy(dst_hbm.at[s, j + off], idx_v.at[q, 1],
                                  isems.at[q]).wait()

        for k in range(PDIST):
            load_idx(k, k)
        for m in range(GDIST):
            wait_idx(m, m)
            pltpu.async_copy(
                hs_hbm.at[idx_v.at[m, 0]], rows_v.at[m], gsems.at[m])

        def step(j, _):
            b = lax.rem(j, NROWBUF)
            q = lax.rem(j, IDX_RING)
            pltpu.make_async_copy(
                hs_hbm.at[idx_v.at[q, 0]], rows_v.at[b], gsems.at[b]).wait()
            pltpu.async_copy(
                rows_v.at[b], acc.at[idx_v.at[q, 1]], ssems.at[b], add=True)

            @pl.when(j + GDIST < kc)
            def _():
                bn = lax.rem(j + GDIST, NROWBUF)
                qn = lax.rem(j + GDIST, IDX_RING)
                @pl.when(j >= 1)
                def _():
                    pltpu.make_async_copy(
                        rows_v.at[bn],
                        acc.at[idx_v.at[lax.rem(j - 1, IDX_RING), 1]],
                        ssems.at[bn]).wait()
                wait_idx(j + GDIST, qn)
                pltpu.async_copy(
                    hs_hbm.at[idx_v.at[qn, 0]], rows_v.at[bn], gsems.at[bn])

                @pl.when(j + PDIST < kc)
                def _():
                    qp = lax.rem(j + PDIST, IDX_RING)
                    load_idx(j + PDIST, qp)
            return 0
        lax.fori_loop(0, kc, step, 0)

        for t in range(GDIST + 1, 0, -1):
            m = kc - t
            pltpu.make_async_copy(
                rows_v.at[lax.rem(m, NROWBUF)],
                acc.at[idx_v.at[lax.rem(m, IDX_RING), 1]],
                ssems.at[lax.rem(m, NROWBUF)]).wait()

        plsc.subcore_barrier()
        for base in ZBASES:
            sl = pl.ds(s * ACC_PER_TILE + base, CHUNK)
            pltpu.sync_copy(acc.at[sl], out_hbm.at[c, sl])

    return sc_agg


_sc_agg128 = _make_sc_agg(D_H)


ROW_BLK = 1024
GRID = N_PAD // ROW_BLK


def _norm_from_deg(dref):
    d = dref[:, 0:1] + dref[:, 1:2]
    return jnp.where(d > 0, lax.rsqrt(jnp.maximum(d, 1.0)), 0.0)


def _tc_h_body(x_ref, w_ref, b_ref, h_ref):
    h_ref[...] = jnp.dot(x_ref[...], w_ref[...],
                         preferred_element_type=jnp.float32) + b_ref[...]


def _tc_h(x, w, b):
    return pl.pallas_call(
        _tc_h_body,
        grid=(GRID,),
        in_specs=[
            pl.BlockSpec((ROW_BLK, D_H), lambda i: (i, 0)),
            pl.BlockSpec((D_H, D_H), lambda i: (0, 0)),
            pl.BlockSpec((1, D_H), lambda i: (0, 0)),
        ],
        out_specs=pl.BlockSpec((ROW_BLK, D_H), lambda i: (i, 0)),
        out_shape=jax.ShapeDtypeStruct((N_PAD, D_H), jnp.float32),
    )(x, w, b)


def _tc_norms_body(h_ref, dsrc_ref, ddst_ref, hs0_ref, ns_ref, nd_ref):
    ns = _norm_from_deg(dsrc_ref)
    nd = _norm_from_deg(ddst_ref)
    hs0_ref[...] = h_ref[...] * ns
    ns_ref[...] = ns
    nd_ref[...] = nd


def _tc_norms(h, dsrc, ddst):
    return pl.pallas_call(
        _tc_norms_body,
        grid=(GRID,),
        in_specs=[
            pl.BlockSpec((ROW_BLK, D_H), lambda i: (i, 0)),
            pl.BlockSpec((ROW_BLK, 2), lambda i: (i, 0)),
            pl.BlockSpec((ROW_BLK, 2), lambda i: (i, 0)),
        ],
        out_specs=[
            pl.BlockSpec((ROW_BLK, D_H), lambda i: (i, 0)),
            pl.BlockSpec((ROW_BLK, 1), lambda i: (i, 0)),
            pl.BlockSpec((ROW_BLK, 1), lambda i: (i, 0)),
        ],
        out_shape=[
            jax.ShapeDtypeStruct((N_PAD, D_H), jnp.float32),
            jax.ShapeDtypeStruct((N_PAD, 1), jnp.float32),
            jax.ShapeDtypeStruct((N_PAD, 1), jnp.float32),
        ],
    )(h, dsrc, ddst)


def _tc_layer_body(p_ref, nd_ref, ns_ref, w_ref, b_ref, out_ref):
    agg = (p_ref[0] + p_ref[1]) * nd_ref[...]
    h = jnp.dot(agg, w_ref[...], preferred_element_type=jnp.float32) + b_ref[...]
    h = jnp.maximum(h, 0.0)
    out_ref[...] = h * ns_ref[...]


def _tc_layer(p, nd, ns, w, b):
    return pl.pallas_call(
        _tc_layer_body,
        grid=(GRID,),
        in_specs=[
            pl.BlockSpec((NC, ROW_BLK, D_H), lambda i: (0, i, 0)),
            pl.BlockSpec((ROW_BLK, 1), lambda i: (i, 0)),
            pl.BlockSpec((ROW_BLK, 1), lambda i: (i, 0)),
            pl.BlockSpec((D_H, D_H), lambda i: (0, 0)),
            pl.BlockSpec((1, D_H), lambda i: (0, 0)),
        ],
        out_specs=pl.BlockSpec((ROW_BLK, D_H), lambda i: (i, 0)),
        out_shape=jax.ShapeDtypeStruct((N_PAD, D_H), jnp.float32),
    )(p, nd, ns, w, b)


def _tc_final_body(p_ref, nd_ref, w2_ref, b2_ref, out_ref):
    agg = (p_ref[0] + p_ref[1]) * nd_ref[...]
    out_ref[...] = jnp.dot(agg, w2_ref[...],
                           preferred_element_type=jnp.float32) + b2_ref[...]


def _tc_final(p, nd, w2, b2):
    return pl.pallas_call(
        _tc_final_body,
        grid=(GRID,),
        in_specs=[
            pl.BlockSpec((NC, ROW_BLK, D_H), lambda i: (0, i, 0)),
            pl.BlockSpec((ROW_BLK, 1), lambda i: (i, 0)),
            pl.BlockSpec((D_H, D_OUT), lambda i: (0, 0)),
            pl.BlockSpec((1, D_OUT), lambda i: (0, 0)),
        ],
        out_specs=pl.BlockSpec((ROW_BLK, D_OUT), lambda i: (i, 0)),
        out_shape=jax.ShapeDtypeStruct((N_PAD, D_OUT), jnp.float32),
    )(p, nd, w2, b2)


def kernel(features, edge_index, W_lin, b_lin, W0, b0, W1, b1, W2, b2):
    pad_e = E_PAD - E

    def slabs(v):
        v = jnp.concatenate([v, jnp.full((pad_e,), DUMMY, jnp.int32)])
        return v.reshape(NS, KT, CHUNK)

    src = slabs(edge_index[0])
    dst = slabs(edge_index[1])

    x = jnp.concatenate(
        [features, jnp.zeros((N_PAD - N, features.shape[1]), jnp.float32)])

    degs = _sc_degrees(src, dst)
    dsrc = degs[:, 0, :].T
    ddst = degs[:, 1, :].T

    h = _tc_h(x, W_lin, b_lin.reshape(1, D_H))
    hs0, ns, nd = _tc_norms(h, dsrc, ddst)

    p0 = _sc_agg128(hs0, src, dst)
    hs1 = _tc_layer(p0, nd, ns, W0, b0.reshape(1, D_H))
    p1 = _sc_agg128(hs1, src, dst)
    hs2 = _tc_layer(p1, nd, ns, W1, b1.reshape(1, D_H))
    p2 = _sc_agg128(hs2, src, dst)
    out = _tc_final(p2, nd, W2, b2.reshape(1, D_OUT))
    return out[:N]


# speedup vs baseline: 1.2271x; 1.0040x over previous
import functools

import jax
import jax.numpy as jnp
from jax import lax
from jax.experimental import pallas as pl
from jax.experimental.pallas import tpu as pltpu
from jax.experimental.pallas import tpu_sc as plsc

N = 10000
N_PAD = 10240
ROWS_PER_TILE = 640
ACC_ROWS = 10240
ACC_PER_TILE = 640
DUMMY = N
E = 320000
CHUNK = 120
NC, NS = 2, 16
NW = NC * NS
K0 = 98
K1 = 69
KT = K0 + K1
E_PAD = NS * KT * CHUNK
ZBASES = (0, 120, 240, 360, 480, 520)
D_H = 128
D_OUT = 64

_mesh = plsc.VectorSubcoreMesh(core_axis_name="c", subcore_axis_name="s")


def _zero_vmem_2d(buf, rows, cols):
    def body(r, _):
        for k in range(cols // 16):
            buf[r, pl.ds(16 * k, 16)] = jnp.zeros((16,), jnp.float32)
        return 0
    lax.fori_loop(0, rows, body, 0)


@functools.partial(
    pl.kernel,
    out_type=jax.ShapeDtypeStruct((NC, 2, N_PAD), jnp.float32),
    mesh=_mesh,
    scratch_types=[
        pltpu.VMEM((KT, CHUNK), jnp.int32),
        pltpu.VMEM((KT, CHUNK), jnp.int32),
        pltpu.VMEM((CHUNK,), jnp.float32),
        pltpu.VMEM((ROWS_PER_TILE,), jnp.float32),
        pltpu.VMEM_SHARED((N_PAD,), jnp.float32),
        pltpu.VMEM_SHARED((N_PAD,), jnp.float32),
        pltpu.SemaphoreType.DMA((4,)),
        pltpu.SemaphoreType.DMA((4,)),
    ],
)
def _sc_degrees(src_hbm, dst_hbm, out_hbm, src_v, dst_v, ones_v, zero_v,
                acc_s, acc_d, ssot, ssod):
    c = lax.axis_index("c")
    s = lax.axis_index("s")
    for k in range(CHUNK // 16):
        ones_v[pl.ds(16 * k, 16)] = jnp.ones((16,), jnp.float32)
    if CHUNK % 16:
        ones_v[pl.ds(CHUNK - 16, 16)] = jnp.ones((16,), jnp.float32)
    for k in range(ROWS_PER_TILE // 16):
        zero_v[pl.ds(16 * k, 16)] = jnp.zeros((16,), jnp.float32)
    pltpu.sync_copy(zero_v, acc_s.at[pl.ds(s * ROWS_PER_TILE, ROWS_PER_TILE)])
    pltpu.sync_copy(zero_v, acc_d.at[pl.ds(s * ROWS_PER_TILE, ROWS_PER_TILE)])
    plsc.subcore_barrier()

    pltpu.sync_copy(src_hbm.at[s], src_v)
    pltpu.sync_copy(dst_hbm.at[s], dst_v)
    off = jnp.where(c == 0, 0, K0)
    kc = jnp.where(c == 0, K0, K1)

    def add(j, _):
        t = lax.rem(j, 4)
        r = j + off

        @pl.when(j >= 4)
        def _():
            pltpu.make_async_copy(
                ones_v, acc_s.at[src_v.at[r - 4]], ssot.at[t]).wait()
            pltpu.make_async_copy(
                ones_v, acc_d.at[dst_v.at[r - 4]], ssod.at[t]).wait()
        pltpu.async_copy(ones_v, acc_s.at[src_v.at[r]], ssot.at[t],
                         add=True)
        pltpu.async_copy(ones_v, acc_d.at[dst_v.at[r]], ssod.at[t],
                         add=True)
        return 0
    lax.fori_loop(0, kc, add, 0)
    for t in range(4, 0, -1):
        m = kc - t + off
        pltpu.make_async_copy(
            ones_v, acc_s.at[src_v.at[m]],
            ssot.at[lax.rem(m - off, 4)]).wait()
        pltpu.make_async_copy(
            ones_v, acc_d.at[dst_v.at[m]],
            ssod.at[lax.rem(m - off, 4)]).wait()

    plsc.subcore_barrier()
    sl = pl.ds(s * ROWS_PER_TILE, ROWS_PER_TILE)
    pltpu.sync_copy(acc_s.at[sl], out_hbm.at[c, 0, sl])
    pltpu.sync_copy(acc_d.at[sl], out_hbm.at[c, 1, sl])


IDX_RING = 5
NROWBUF = 3
GDIST = NROWBUF - 1
PDIST = IDX_RING - 1


def _make_sc_agg(d):
    @functools.partial(
        pl.kernel,
        out_type=jax.ShapeDtypeStruct((NC, N_PAD, d), jnp.float32),
        mesh=_mesh,
        scratch_types=[
            pltpu.VMEM((IDX_RING, 2, CHUNK), jnp.int32),
            pltpu.VMEM((NROWBUF, CHUNK, d), jnp.float32),
            pltpu.VMEM_SHARED((ACC_ROWS, d), jnp.float32),
            pltpu.SemaphoreType.DMA((IDX_RING,)),
            pltpu.SemaphoreType.DMA((NROWBUF,)),
            pltpu.SemaphoreType.DMA((NROWBUF,)),
        ],
    )
    def sc_agg(hs_hbm, src_hbm, dst_hbm, out_hbm, idx_v, rows_v, acc,
               isems, gsems, ssems):
        c = lax.axis_index("c")
        s = lax.axis_index("s")
        _zero_vmem_2d(rows_v.at[0], CHUNK, d)
        for base in ZBASES:
            pltpu.sync_copy(
                rows_v.at[0], acc.at[pl.ds(s * ACC_PER_TILE + base, CHUNK)])
        plsc.subcore_barrier()

        off = jnp.where(c == 0, 0, K0)
        kc = jnp.where(c == 0, K0, K1)

        def load_idx(j, q):
            pltpu.async_copy(src_hbm.at[s, j + off], idx_v.at[q, 0],
                             isems.at[q])
            pltpu.async_copy(dst_hbm.at[s, j + off], idx_v.at[q, 1],
                             isems.at[q])

        def wait_idx(j, q):
            pltpu.make_async_copy(src_hbm.at[s, j + off], idx_v.at[q, 0],
                                  isems.at[q]).wait()
            pltpu.make_async_copy(dst_hbm.at[s, j + off], idx_v.at[q, 1],
                                  isems.at[q]).wait()

        for k in range(PDIST):
            load_idx(k, k)
        for m in range(GDIST):
            wait_idx(m, m)
            pltpu.async_copy(
                hs_hbm.at[idx_v.at[m, 0]], rows_v.at[m], gsems.at[m])

        def step(j, _):
            b = lax.rem(j, NROWBUF)
            q = lax.rem(j, IDX_RING)
            pltpu.make_async_copy(
                hs_hbm.at[idx_v.at[q, 0]], rows_v.at[b], gsems.at[b]).wait()
            pltpu.async_copy(
                rows_v.at[b], acc.at[idx_v.at[q, 1]], ssems.at[b], add=True)

            @pl.when(j + GDIST < kc)
            def _():
                bn = lax.rem(j + GDIST, NROWBUF)
                qn = lax.rem(j + GDIST, IDX_RING)
                @pl.when(j >= 1)
                def _():
                    pltpu.make_async_copy(
                        rows_v.at[bn],
                        acc.at[idx_v.at[lax.rem(j - 1, IDX_RING), 1]],
                        ssems.at[bn]).wait()
                wait_idx(j + GDIST, qn)
                pltpu.async_copy(
                    hs_hbm.at[idx_v.at[qn, 0]], rows_v.at[bn], gsems.at[bn])

                @pl.when(j + PDIST < kc)
                def _():
                    qp = lax.rem(j + PDIST, IDX_RING)
                    load_idx(j + PDIST, qp)
            return 0
        lax.fori_loop(0, kc, step, 0)

        for t in range(GDIST + 1, 0, -1):
            m = kc - t
            pltpu.make_async_copy(
                rows_v.at[lax.rem(m, NROWBUF)],
                acc.at[idx_v.at[lax.rem(m, IDX_RING), 1]],
                ssems.at[lax.rem(m, NROWBUF)]).wait()

        plsc.subcore_barrier()
        for base in ZBASES:
            sl = pl.ds(s * ACC_PER_TILE + base, CHUNK)
            pltpu.sync_copy(acc.at[sl], out_hbm.at[c, sl])

    return sc_agg


_sc_agg128 = _make_sc_agg(D_H)


ROW_BLK = 1024
GRID = N_PAD // ROW_BLK


def _norm_from_deg(dref):
    d = dref[:, 0:1] + dref[:, 1:2]
    return jnp.where(d > 0, lax.rsqrt(jnp.maximum(d, 1.0)), 0.0)


def _tc_h_body(x_ref, w_ref, b_ref, h_ref):
    h_ref[...] = jnp.dot(x_ref[...], w_ref[...],
                         preferred_element_type=jnp.float32) + b_ref[...]


def _tc_h(x, w, b):
    return pl.pallas_call(
        _tc_h_body,
        grid=(GRID,),
        in_specs=[
            pl.BlockSpec((ROW_BLK, D_H), lambda i: (i, 0)),
            pl.BlockSpec((D_H, D_H), lambda i: (0, 0)),
            pl.BlockSpec((1, D_H), lambda i: (0, 0)),
        ],
        out_specs=pl.BlockSpec((ROW_BLK, D_H), lambda i: (i, 0)),
        out_shape=jax.ShapeDtypeStruct((N_PAD, D_H), jnp.float32),
    )(x, w, b)


def _tc_norms_body(h_ref, dsrc_ref, ddst_ref, hs0_ref, ns_ref, nd_ref):
    ns = _norm_from_deg(dsrc_ref)
    nd = _norm_from_deg(ddst_ref)
    hs0_ref[...] = h_ref[...] * ns
    ns_ref[...] = ns
    nd_ref[...] = nd


def _tc_norms(h, dsrc, ddst):
    return pl.pallas_call(
        _tc_norms_body,
        grid=(GRID,),
        in_specs=[
            pl.BlockSpec((ROW_BLK, D_H), lambda i: (i, 0)),
            pl.BlockSpec((ROW_BLK, 2), lambda i: (i, 0)),
            pl.BlockSpec((ROW_BLK, 2), lambda i: (i, 0)),
        ],
        out_specs=[
            pl.BlockSpec((ROW_BLK, D_H), lambda i: (i, 0)),
            pl.BlockSpec((ROW_BLK, 1), lambda i: (i, 0)),
            pl.BlockSpec((ROW_BLK, 1), lambda i: (i, 0)),
        ],
        out_shape=[
            jax.ShapeDtypeStruct((N_PAD, D_H), jnp.float32),
            jax.ShapeDtypeStruct((N_PAD, 1), jnp.float32),
            jax.ShapeDtypeStruct((N_PAD, 1), jnp.float32),
        ],
    )(h, dsrc, ddst)


def _tc_layer_body(p_ref, nd_ref, ns_ref, w_ref, b_ref, out_ref):
    agg = (p_ref[0] + p_ref[1]) * nd_ref[...]
    h = jnp.dot(agg, w_ref[...], preferred_element_type=jnp.float32) + b_ref[...]
    h = jnp.maximum(h, 0.0)
    out_ref[...] = h * ns_ref[...]


def _tc_layer(p, nd, ns, w, b):
    return pl.pallas_call(
        _tc_layer_body,
        grid=(GRID,),
        in_specs=[
            pl.BlockSpec((NC, ROW_BLK, D_H), lambda i: (0, i, 0)),
            pl.BlockSpec((ROW_BLK, 1), lambda i: (i, 0)),
            pl.BlockSpec((ROW_BLK, 1), lambda i: (i, 0)),
            pl.BlockSpec((D_H, D_H), lambda i: (0, 0)),
            pl.BlockSpec((1, D_H), lambda i: (0, 0)),
        ],
        out_specs=pl.BlockSpec((ROW_BLK, D_H), lambda i: (i, 0)),
        out_shape=jax.ShapeDtypeStruct((N_PAD, D_H), jnp.float32),
    )(p, nd, ns, w, b)


def _tc_final_body(p_ref, nd_ref, w2_ref, b2_ref, out_ref):
    agg = (p_ref[0] + p_ref[1]) * nd_ref[...]
    out_ref[...] = jnp.dot(agg, w2_ref[...],
                           preferred_element_type=jnp.float32) + b2_ref[...]


def _tc_final(p, nd, w2, b2):
    return pl.pallas_call(
        _tc_final_body,
        grid=(GRID,),
        in_specs=[
            pl.BlockSpec((NC, ROW_BLK, D_H), lambda i: (0, i, 0)),
            pl.BlockSpec((ROW_BLK, 1), lambda i: (i, 0)),
            pl.BlockSpec((D_H, D_OUT), lambda i: (0, 0)),
            pl.BlockSpec((1, D_OUT), lambda i: (0, 0)),
        ],
        out_specs=pl.BlockSpec((ROW_BLK, D_OUT), lambda i: (i, 0)),
        out_shape=jax.ShapeDtypeStruct((N_PAD, D_OUT), jnp.float32),
    )(p, nd, w2, b2)


def kernel(features, edge_index, W_lin, b_lin, W0, b0, W1, b1, W2, b2):
    pad_e = E_PAD - E

    def slabs(v):
        v = jnp.concatenate([v, jnp.full((pad_e,), DUMMY, jnp.int32)])
        return v.reshape(NS, KT, CHUNK)

    src = slabs(edge_index[0])
    dst = slabs(edge_index[1])

    x = jnp.concatenate(
        [features, jnp.zeros((N_PAD - N, features.shape[1]), jnp.float32)])

    degs = _sc_degrees(src, dst)
    dsrc = degs[:, 0, :].T
    ddst = degs[:, 1, :].T

    h = _tc_h(x, W_lin, b_lin.reshape(1, D_H))
    hs0, ns, nd = _tc_norms(h, dsrc, ddst)

    p0 = _sc_agg128(hs0, src, dst)
    hs1 = _tc_layer(p0, nd, ns, W0, b0.reshape(1, D_H))
    p1 = _sc_agg128(hs1, src, dst)
    hs2 = _tc_layer(p1, nd, ns, W1, b1.reshape(1, D_H))
    p2 = _sc_agg128(hs2, src, dst)
    out = _tc_final(p2, nd, W2, b2.reshape(1, D_OUT))
    return out[:N]
